# Initial kernel scaffold; baseline (speedup 1.0000x reference)
import jax
import jax.numpy as jnp
from jax.experimental import pallas as pl


def kernel(node_features, edge_index, init_config, target_config, W1, b1, W2, b2, Wn1, bn1, Wn2, bn2, Wa1, ba1, Wa2, ba2, Wc, bc):
    raise NotImplementedError("write your pallas kernel here")


# speedup vs baseline: 16.8305x; 16.8305x over previous
import functools

import jax
import jax.numpy as jnp
from jax import lax
from jax.experimental import pallas as pl
from jax.experimental.pallas import tpu as pltpu

N = 30
E = 120
NF = 9
L = 10
FIN = 3
H = 12
UROWS = NF * (N + 1)
ZROWS = 2 * NF + 1


def _build_tables_jax(edge_index, init_config, target_config, node_features):
    Bg = edge_index.shape[0]
    src = edge_index[:, 0, :]
    dst = edge_index[:, 1, :]
    cell = dst * N + src
    bidx = jnp.arange(Bg, dtype=jnp.int32)[:, None]
    C = jnp.zeros((Bg, N * N), jnp.float32).at[bidx, cell].add(1.0)
    Ui = jnp.zeros((Bg, UROWS), jnp.float32).at[
        bidx, (jnp.arange(NF, dtype=jnp.int32)[:, None] * (N + 1)
               + init_config.astype(jnp.int32)).reshape(Bg, NF * L)].add(1.0)
    Ut = jnp.zeros((Bg, UROWS), jnp.float32).at[
        bidx, (jnp.arange(NF, dtype=jnp.int32)[:, None] * (N + 1)
               + target_config.astype(jnp.int32)).reshape(Bg, NF * L)].add(1.0)
    xt = node_features.reshape(Bg, N * FIN).T
    return C.T, Ui.T, Ut.T, xt


def _tc_kernel(C_ref, xt_ref, Ui_ref, Ut_ref,
               W1_ref, b1_ref, W2_ref, b2_ref,
               Wn1_ref, bn1_ref, Wn2_ref, bn2_ref,
               Wa1_ref, ba1_ref, Wa2_ref, ba2_ref, Wc_ref, bc_ref,
               out_ref, dinv_ref, y_ref, h_ref, z_ref):
    f32 = jnp.float32
    zero = jnp.zeros((8, 128), f32)

    for n in range(N):
        deg = lax.fori_loop(
            0, N, lambda m, a, n=n: a + C_ref[n * N + m],
            jnp.full((8, 128), 1.0, f32))
        dinv_ref[n] = lax.rsqrt(deg)

    for n in range(N):
        xk = [xt_ref[n * FIN + k] for k in range(FIN)]
        dn = dinv_ref[n]
        for j in range(H):
            hp = xk[0] * W1_ref[0, j]
            for k in range(1, FIN):
                hp = hp + xk[k] * W1_ref[k, j]
            y_ref[n * H + j] = dn * hp

    for n in range(N):
        def inner1(m, acc, n=n):
            c = C_ref[n * N + m]
            return [acc[j] + c * y_ref[m * H + j] for j in range(H)]
        acc = lax.fori_loop(0, N, inner1,
                            [y_ref[n * H + j] for j in range(H)])
        dn = dinv_ref[n]
        for j in range(H):
            h_ref[n * H + j] = jnp.maximum(dn * acc[j] + b1_ref[j], 0.0)

    for n in range(N):
        hk = [h_ref[n * H + k] for k in range(H)]
        dn = dinv_ref[n]
        for j in range(H):
            hp = hk[0] * W2_ref[0, j]
            for k in range(1, H):
                hp = hp + hk[k] * W2_ref[k, j]
            y_ref[n * H + j] = dn * hp

    for n in range(N):
        def inner2(m, acc, n=n):
            c = C_ref[n * N + m]
            return [acc[j] + c * y_ref[m * H + j] for j in range(H)]
        acc = lax.fori_loop(0, N, inner2,
                            [y_ref[n * H + j] for j in range(H)])
        dn = dinv_ref[n]
        for j in range(H):
            h_ref[n * H + j] = dn * acc[j] + b2_ref[j]

    for j in range(H):
        s = lax.fori_loop(1, N, lambda n, a, j=j: a + h_ref[n * H + j],
                          h_ref[j])
        z_ref[j] = s * (1.0 / N)

    for base, U_ref in ((1, Ui_ref), (1 + NF, Ut_ref)):
        for f in range(NF):
            def innerU(v, acc, f=f, U_ref=U_ref):
                u = U_ref[f * (N + 1) + v]
                return [acc[j] + u * h_ref[(v - 1) * H + j] for j in range(H)]
            acc = lax.fori_loop(1, N + 1, innerU, [zero] * H)
            for j in range(H):
                z_ref[(base + f) * H + j] = acc[j]

    def mlp_body(r, accm):
        zr = [z_ref[r * H + k] for k in range(H)]
        t1 = []
        for j in range(H):
            s = zr[0] * Wn1_ref[0, j]
            for k in range(1, H):
                s = s + zr[k] * Wn1_ref[k, j]
            t1.append(jnp.maximum(s + bn1_ref[j], 0.0))
        out = []
        for j in range(H):
            s = t1[0] * Wn2_ref[0, j]
            for k in range(1, H):
                s = s + t1[k] * Wn2_ref[k, j]
            out.append(accm[j] + s + bn2_ref[j])
        return out
    accm = lax.fori_loop(0, ZROWS, mlp_body, [zero] * H)
    zm = [a * (1.0 / ZROWS) for a in accm]

    a1 = []
    for j in range(H):
        s = zm[0] * Wa1_ref[0, j]
        for k in range(1, H):
            s = s + zm[k] * Wa1_ref[k, j]
        a1.append(jnp.maximum(s + ba1_ref[j], 0.0))
    for p in range(NF):
        s = a1[0] * Wa2_ref[0, p]
        for k in range(1, H):
            s = s + a1[k] * Wa2_ref[k, p]
        out_ref[p] = s + ba2_ref[p]
    s = zm[0] * Wc_ref[0, 0]
    for k in range(1, H):
        s = s + zm[k] * Wc_ref[k, 0]
    out_ref[NF] = s + bc_ref[0]


def _run_tc(C2, Ui2, Ut2, xt2, W1, b1, W2, b2, Wn1, bn1, Wn2, bn2,
            Wa1, ba1, Wa2, ba2, Wc, bc, interpret=False):
    Bg = C2.shape[1]
    nblk = Bg // 1024
    C3 = C2.reshape(N * N, Bg // 128, 128)
    Ui3 = Ui2.reshape(UROWS, Bg // 128, 128)
    Ut3 = Ut2.reshape(UROWS, Bg // 128, 128)
    xt3 = xt2.reshape(N * FIN, Bg // 128, 128)

    def vspec(rows):
        return pl.BlockSpec((rows, 8, 128), lambda i: (0, i, 0))
    sspec = pl.BlockSpec(memory_space=pltpu.SMEM)

    out = pl.pallas_call(
        _tc_kernel,
        grid=(nblk,),
        in_specs=[vspec(N * N), vspec(N * FIN), vspec(UROWS), vspec(UROWS)]
                 + [sspec] * 14,
        out_specs=pl.BlockSpec((NF + 1, 8, 128), lambda i: (0, i, 0)),
        out_shape=jax.ShapeDtypeStruct((NF + 1, Bg // 128, 128), jnp.float32),
        scratch_shapes=[
            pltpu.VMEM((N, 8, 128), jnp.float32),
            pltpu.VMEM((N * H, 8, 128), jnp.float32),
            pltpu.VMEM((N * H, 8, 128), jnp.float32),
            pltpu.VMEM((ZROWS * H, 8, 128), jnp.float32),
        ],
        interpret=interpret,
    )(C3, xt3, Ui3, Ut3, W1, b1, W2, b2, Wn1, bn1, Wn2, bn2,
      Wa1, ba1, Wa2, ba2, Wc, bc)
    out2 = out.reshape(NF + 1, Bg)
    return out2[:NF].T, out2[NF:NF + 1].T


def kernel(node_features, edge_index, init_config, target_config,
           W1, b1, W2, b2, Wn1, bn1, Wn2, bn2, Wa1, ba1, Wa2, ba2, Wc, bc):
    C2, Ui2, Ut2, xt2 = _build_tables_jax(
        edge_index, init_config, target_config, node_features)
    logits, value = _run_tc(C2, Ui2, Ut2, xt2, W1, b1, W2, b2,
                            Wn1, bn1, Wn2, bn2, Wa1, ba1, Wa2, ba2, Wc, bc)
    return (logits, value)


# speedup vs baseline: 145.8717x; 8.6671x over previous
import functools

import jax
import jax.numpy as jnp
from jax import lax
from jax.experimental import pallas as pl
from jax.experimental.pallas import tpu as pltpu
from jax.experimental.pallas import tpu_sc as plsc

N = 30
E = 120
NF = 9
L = 10
FIN = 3
H = 12
UROWS = NF * (N + 1)
ZROWS = 2 * NF + 1


def _build_tables_jax(edge_index, init_config, target_config, node_features):
    Bg = edge_index.shape[0]
    src = edge_index[:, 0, :]
    dst = edge_index[:, 1, :]
    cell = dst * N + src
    bidx = jnp.arange(Bg, dtype=jnp.int32)[:, None]
    C = jnp.zeros((Bg, N * N), jnp.float32).at[bidx, cell].add(1.0)
    Ui = jnp.zeros((Bg, UROWS), jnp.float32).at[
        bidx, (jnp.arange(NF, dtype=jnp.int32)[:, None] * (N + 1)
               + init_config.astype(jnp.int32)).reshape(Bg, NF * L)].add(1.0)
    Ut = jnp.zeros((Bg, UROWS), jnp.float32).at[
        bidx, (jnp.arange(NF, dtype=jnp.int32)[:, None] * (N + 1)
               + target_config.astype(jnp.int32)).reshape(Bg, NF * L)].add(1.0)
    xt = node_features.reshape(Bg, N * FIN).T
    return C.T, Ui.T, Ut.T, xt


def _sc_build_tables(edge_index, init_config, target_config, node_features):
    Bg = edge_index.shape[0]
    info = plsc.get_sparse_core_info()
    NC, NS, LN = info.num_cores, info.num_subcores, info.num_lanes
    NW = NC * NS
    GP = Bg // (NW * LN)
    mesh = plsc.VectorSubcoreMesh(core_axis_name="c", subcore_axis_name="s")

    CW = N * N
    EW = 2 * E
    FW = NF * L

    @functools.partial(
        pl.kernel, mesh=mesh,
        compiler_params=pltpu.CompilerParams(needs_layout_passes=False),
        out_type=[
            jax.ShapeDtypeStruct((Bg * CW,), jnp.float32),
            jax.ShapeDtypeStruct((Bg * UROWS,), jnp.float32),
            jax.ShapeDtypeStruct((Bg * UROWS,), jnp.float32),
        ],
        scratch_types=[
            pltpu.VMEM((LN * EW,), jnp.int32),
            pltpu.VMEM((LN * FW,), jnp.int32),
            pltpu.VMEM((LN * FW,), jnp.int32),
            pltpu.VMEM((LN * CW,), jnp.float32),
            pltpu.VMEM((LN * UROWS,), jnp.float32),
            pltpu.VMEM((LN * UROWS,), jnp.float32),
        ],
    )
    def build(edge_hbm, init_hbm, targ_hbm,
              C_hbm, Ui_hbm, Ut_hbm,
              ebuf, ibuf, tbuf, cbuf, uibuf, utbuf):
        i32 = jnp.int32
        f32 = jnp.float32
        iota = lax.iota(i32, LN)
        ones = jnp.ones((LN,), f32)
        zf = jnp.zeros((LN,), f32)
        iota_e = iota * EW
        iota_f = iota * FW
        iota_c = iota * CW
        iota_u = iota * UROWS
        wid = lax.axis_index("s") * NC + lax.axis_index("c")

        def group_body(g, _):
            g0 = (wid * GP + g) * LN
            pltpu.sync_copy(edge_hbm.at[pl.ds(g0 * EW, LN * EW)], ebuf)
            pltpu.sync_copy(init_hbm.at[pl.ds(g0 * FW, LN * FW)], ibuf)
            pltpu.sync_copy(targ_hbm.at[pl.ds(g0 * FW, LN * FW)], tbuf)

            def zc(r, _):
                plsc.store_scatter(cbuf, [iota_c + r], zf)
                return ()
            lax.fori_loop(0, CW, zc, (), unroll=8)

            def zu(r, _):
                plsc.store_scatter(uibuf, [iota_u + r], zf)
                plsc.store_scatter(utbuf, [iota_u + r], zf)
                return ()
            lax.fori_loop(0, UROWS, zu, (), unroll=8)

            def ed(e, _):
                vsrc = plsc.load_gather(ebuf, [iota_e + e])
                vdst = plsc.load_gather(ebuf, [iota_e + (E + e)])
                plsc.addupdate_scatter(
                    cbuf, [iota_c + (vdst * N + vsrc)], ones)
                return ()
            lax.fori_loop(0, E, ed, (), unroll=4)

            for cfg, ubuf in ((ibuf, uibuf), (tbuf, utbuf)):
                def fl(f, _, cfg=cfg, ubuf=ubuf):
                    base = f * (N + 1)
                    def ll(l, __, cfg=cfg, ubuf=ubuf, base=base, f=f):
                        v = plsc.load_gather(cfg, [iota_f + (f * L + l)])
                        plsc.addupdate_scatter(
                            ubuf, [iota_u + (base + v)], ones)
                        return ()
                    lax.fori_loop(0, L, ll, (), unroll=5)
                    return ()
                lax.fori_loop(0, NF, fl, ())

            pltpu.sync_copy(cbuf, C_hbm.at[pl.ds(g0 * CW, LN * CW)])
            pltpu.sync_copy(uibuf, Ui_hbm.at[pl.ds(g0 * UROWS, LN * UROWS)])
            pltpu.sync_copy(utbuf, Ut_hbm.at[pl.ds(g0 * UROWS, LN * UROWS)])
            return ()

        lax.fori_loop(0, GP, group_body, ())

    Cf, Uif, Utf = build(edge_index.reshape(Bg * EW),
                         init_config.reshape(Bg * FW),
                         target_config.reshape(Bg * FW))
    return Cf.reshape(Bg, CW), Uif.reshape(Bg, UROWS), Utf.reshape(Bg, UROWS)


def _tc_kernel(C_ref, xt_ref, Ui_ref, Ut_ref,
               W1_ref, b1_ref, W2_ref, b2_ref,
               Wn1_ref, bn1_ref, Wn2_ref, bn2_ref,
               Wa1_ref, ba1_ref, Wa2_ref, ba2_ref, Wc_ref, bc_ref,
               out_ref, dinv_ref, y_ref, h_ref, z_ref):
    f32 = jnp.float32
    zero = jnp.zeros((8, 128), f32)

    for n in range(N):
        deg = lax.fori_loop(
            0, N, lambda m, a, n=n: a + C_ref[n * N + m],
            jnp.full((8, 128), 1.0, f32))
        dinv_ref[n] = lax.rsqrt(deg)

    for n in range(N):
        xk = [xt_ref[n * FIN + k] for k in range(FIN)]
        dn = dinv_ref[n]
        for j in range(H):
            hp = xk[0] * W1_ref[0, j]
            for k in range(1, FIN):
                hp = hp + xk[k] * W1_ref[k, j]
            y_ref[n * H + j] = dn * hp

    for n in range(N):
        def inner1(m, acc, n=n):
            c = C_ref[n * N + m]
            return [acc[j] + c * y_ref[m * H + j] for j in range(H)]
        acc = lax.fori_loop(0, N, inner1,
                            [y_ref[n * H + j] for j in range(H)])
        dn = dinv_ref[n]
        for j in range(H):
            h_ref[n * H + j] = jnp.maximum(dn * acc[j] + b1_ref[j], 0.0)

    for n in range(N):
        hk = [h_ref[n * H + k] for k in range(H)]
        dn = dinv_ref[n]
        for j in range(H):
            hp = hk[0] * W2_ref[0, j]
            for k in range(1, H):
                hp = hp + hk[k] * W2_ref[k, j]
            y_ref[n * H + j] = dn * hp

    for n in range(N):
        def inner2(m, acc, n=n):
            c = C_ref[n * N + m]
            return [acc[j] + c * y_ref[m * H + j] for j in range(H)]
        acc = lax.fori_loop(0, N, inner2,
                            [y_ref[n * H + j] for j in range(H)])
        dn = dinv_ref[n]
        for j in range(H):
            h_ref[n * H + j] = dn * acc[j] + b2_ref[j]

    for j in range(H):
        s = lax.fori_loop(1, N, lambda n, a, j=j: a + h_ref[n * H + j],
                          h_ref[j])
        z_ref[j] = s * (1.0 / N)

    for base, U_ref in ((1, Ui_ref), (1 + NF, Ut_ref)):
        for f in range(NF):
            def innerU(v, acc, f=f, U_ref=U_ref):
                u = U_ref[f * (N + 1) + v]
                return [acc[j] + u * h_ref[(v - 1) * H + j] for j in range(H)]
            acc = lax.fori_loop(1, N + 1, innerU, [zero] * H)
            for j in range(H):
                z_ref[(base + f) * H + j] = acc[j]

    def mlp_body(r, accm):
        zr = [z_ref[r * H + k] for k in range(H)]
        t1 = []
        for j in range(H):
            s = zr[0] * Wn1_ref[0, j]
            for k in range(1, H):
                s = s + zr[k] * Wn1_ref[k, j]
            t1.append(jnp.maximum(s + bn1_ref[j], 0.0))
        out = []
        for j in range(H):
            s = t1[0] * Wn2_ref[0, j]
            for k in range(1, H):
                s = s + t1[k] * Wn2_ref[k, j]
            out.append(accm[j] + s + bn2_ref[j])
        return out
    accm = lax.fori_loop(0, ZROWS, mlp_body, [zero] * H)
    zm = [a * (1.0 / ZROWS) for a in accm]

    a1 = []
    for j in range(H):
        s = zm[0] * Wa1_ref[0, j]
        for k in range(1, H):
            s = s + zm[k] * Wa1_ref[k, j]
        a1.append(jnp.maximum(s + ba1_ref[j], 0.0))
    for p in range(NF):
        s = a1[0] * Wa2_ref[0, p]
        for k in range(1, H):
            s = s + a1[k] * Wa2_ref[k, p]
        out_ref[p] = s + ba2_ref[p]
    s = zm[0] * Wc_ref[0, 0]
    for k in range(1, H):
        s = s + zm[k] * Wc_ref[k, 0]
    out_ref[NF] = s + bc_ref[0]


def _run_tc(C2, Ui2, Ut2, xt2, W1, b1, W2, b2, Wn1, bn1, Wn2, bn2,
            Wa1, ba1, Wa2, ba2, Wc, bc, interpret=False):
    Bg = C2.shape[1]
    nblk = Bg // 1024
    C3 = C2.reshape(N * N, Bg // 128, 128)
    Ui3 = Ui2.reshape(UROWS, Bg // 128, 128)
    Ut3 = Ut2.reshape(UROWS, Bg // 128, 128)
    xt3 = xt2.reshape(N * FIN, Bg // 128, 128)

    def vspec(rows):
        return pl.BlockSpec((rows, 8, 128), lambda i: (0, i, 0))
    sspec = pl.BlockSpec(memory_space=pltpu.SMEM)

    out = pl.pallas_call(
        _tc_kernel,
        grid=(nblk,),
        in_specs=[vspec(N * N), vspec(N * FIN), vspec(UROWS), vspec(UROWS)]
                 + [sspec] * 14,
        out_specs=pl.BlockSpec((NF + 1, 8, 128), lambda i: (0, i, 0)),
        out_shape=jax.ShapeDtypeStruct((NF + 1, Bg // 128, 128), jnp.float32),
        scratch_shapes=[
            pltpu.VMEM((N, 8, 128), jnp.float32),
            pltpu.VMEM((N * H, 8, 128), jnp.float32),
            pltpu.VMEM((N * H, 8, 128), jnp.float32),
            pltpu.VMEM((ZROWS * H, 8, 128), jnp.float32),
        ],
        interpret=interpret,
    )(C3, xt3, Ui3, Ut3, W1, b1, W2, b2, Wn1, bn1, Wn2, bn2,
      Wa1, ba1, Wa2, ba2, Wc, bc)
    out2 = out.reshape(NF + 1, Bg)
    return out2[:NF].T, out2[NF:NF + 1].T


def kernel(node_features, edge_index, init_config, target_config,
           W1, b1, W2, b2, Wn1, bn1, Wn2, bn2, Wa1, ba1, Wa2, ba2, Wc, bc):
    Cgm, Uigm, Utgm = _sc_build_tables(
        edge_index, init_config, target_config, node_features)
    C2 = Cgm.T
    Ui2 = Uigm.T
    Ut2 = Utgm.T
    xt2 = node_features.reshape(node_features.shape[0], N * FIN).T
    logits, value = _run_tc(C2, Ui2, Ut2, xt2, W1, b1, W2, b2,
                            Wn1, bn1, Wn2, bn2, Wa1, ba1, Wa2, ba2, Wc, bc)
    return (logits, value)


# speedup vs baseline: 172.3992x; 1.1819x over previous
import functools

import jax
import jax.numpy as jnp
from jax import lax
from jax.experimental import pallas as pl
from jax.experimental.pallas import tpu as pltpu
from jax.experimental.pallas import tpu_sc as plsc

N = 30
E = 120
NF = 9
L = 10
FIN = 3
H = 12
UROWS = NF * (N + 1)
ZROWS = 2 * NF + 1
CWP = N * N + 1
UI0 = CWP
UT0 = CWP + UROWS
TW = CWP + 2 * UROWS
EW = 2 * E
FW = NF * L


def _sc_build_tables(edge_index, init_config, target_config):
    Bg = edge_index.shape[0]
    info = plsc.get_sparse_core_info()
    NC, NS, LN = info.num_cores, info.num_subcores, info.num_lanes
    NW = NC * NS
    gper = Bg // NW
    SGRP = 4
    rounds = gper // (SGRP * LN)
    mesh = plsc.VectorSubcoreMesh(core_axis_name="c", subcore_axis_name="s")

    SLOT = LN * TW
    NSAVE = E + 2 * NF * L

    @functools.partial(
        pl.kernel, mesh=mesh,
        compiler_params=pltpu.CompilerParams(needs_layout_passes=False),
        out_type=jax.ShapeDtypeStruct((Bg * TW,), jnp.float32),
        scratch_types=[
            pltpu.VMEM((SGRP * LN * EW,), jnp.int32),
            pltpu.VMEM((SGRP * LN * FW,), jnp.int32),
            pltpu.VMEM((SGRP * LN * FW,), jnp.int32),
            pltpu.VMEM((2 * SLOT,), jnp.float32),
            pltpu.VMEM((2 * NSAVE * LN,), jnp.int32),
        ],
    )
    def build(edge_hbm, init_hbm, targ_hbm, T_hbm,
              ebuf, ibuf, tbuf, work, abuf):
        i32 = jnp.int32
        f32 = jnp.float32
        iota = lax.iota(i32, LN)
        ones = jnp.ones((LN,), f32)
        zf = jnp.zeros((LN,), f32)
        iota_t = iota * TW
        iota_e = iota * EW
        iota_f = iota * FW
        wid = lax.axis_index("s") * NC + lax.axis_index("c")
        w0 = wid * gper

        def z0(r, _):
            plsc.store_scatter(work, [iota_t + r], zf)
            plsc.store_scatter(work, [iota_t + (SLOT + r)], zf)
            return ()
        lax.fori_loop(0, TW, z0, (), unroll=8)

        def round_body(rd, _):
            r0 = w0 + rd * (SGRP * LN)
            pltpu.sync_copy(edge_hbm.at[pl.ds(r0 * EW, SGRP * LN * EW)], ebuf)
            pltpu.sync_copy(init_hbm.at[pl.ds(r0 * FW, SGRP * LN * FW)], ibuf)
            pltpu.sync_copy(targ_hbm.at[pl.ds(r0 * FW, SGRP * LN * FW)], tbuf)

            for gg in range(SGRP):
                s = gg % 2
                sbase = s * SLOT
                abase = s * NSAVE * LN
                se = gg * LN * EW
                sf = gg * LN * FW

                def ed(e, _, sbase=sbase, abase=abase, se=se):
                    eg = iota + e
                    eg = jnp.where(eg >= E, eg - E, eg)
                    vsrc = plsc.load_gather(ebuf, [iota_e + (se + eg)])
                    vdst = plsc.load_gather(
                        ebuf, [iota_e + (se + E + eg)])
                    addr = iota_t + (sbase + (vdst * N + vsrc))
                    plsc.addupdate_scatter(work, [addr], ones)
                    plsc.store_scatter(
                        abuf, [iota + (abase + e * LN)], addr)
                    return ()
                lax.fori_loop(0, E, ed, (), unroll=4)

                for ci, cfg in ((0, ibuf), (1, tbuf)):
                    def fl(f, _, ci=ci, cfg=cfg, sbase=sbase,
                           abase=abase, sf=sf):
                        ub = sbase + UI0 + ci * UROWS + f * (N + 1)
                        asl = abase + (E + ci * FW + f * L) * LN
                        def ll(l, __, cfg=cfg, ub=ub, asl=asl, f=f, sf=sf):
                            v = plsc.load_gather(
                                cfg, [iota_f + (sf + f * L + l)])
                            addr = iota_t + (ub + v)
                            plsc.addupdate_scatter(work, [addr], ones)
                            plsc.store_scatter(
                                abuf, [iota + (asl + l * LN)], addr)
                            return ()
                        lax.fori_loop(0, L, ll, (), unroll=5)
                        return ()
                    lax.fori_loop(0, NF, fl, ())

                if gg % 2 == 1:
                    b0 = r0 + (gg - 1) * LN
                    pltpu.sync_copy(
                        work, T_hbm.at[pl.ds(b0 * TW, 2 * SLOT)])
                    def rz(t, _):
                        av = plsc.load_gather(abuf, [iota + t * LN])
                        plsc.store_scatter(work, [av], zf)
                        return ()
                    lax.fori_loop(0, 2 * NSAVE, rz, (), unroll=8)
            return ()

        lax.fori_loop(0, rounds, round_body, ())

    return build(edge_index.reshape(Bg * EW),
                 init_config.reshape(Bg * FW),
                 target_config.reshape(Bg * FW))


def _tc_kernel(T_ref, xt_ref,
               W1_ref, b1_ref, W2_ref, b2_ref,
               Wn1_ref, bn1_ref, Wn2_ref, bn2_ref,
               Wa1_ref, ba1_ref, Wa2_ref, ba2_ref, Wc_ref, bc_ref,
               out_ref, dinv_ref, y_ref, h_ref, z_ref):
    f32 = jnp.float32
    zero = jnp.zeros((8, 128), f32)
    CH = 3

    for n in range(N):
        s = T_ref[n * N]
        for m in range(1, N):
            s = s + T_ref[n * N + m]
        dinv_ref[n] = 1.0 / jnp.sqrt(s + 1.0)

    for n in range(N):
        xk = [xt_ref[n * FIN + k] for k in range(FIN)]
        dn = dinv_ref[n]
        for j in range(H):
            hp = xk[0] * W1_ref[0, j]
            for k in range(1, FIN):
                hp = hp + xk[k] * W1_ref[k, j]
            y_ref[n * H + j] = dn * hp

    def conv(dst_ref, bias_ref, relu):
        for nc in range(0, N, CH):
            def inner(m, acc, nc=nc):
                ym = [y_ref[m * H + j] for j in range(H)]
                cs = [T_ref[(nc + i) * N + m] for i in range(CH)]
                return tuple(
                    tuple(acc[i][j] + cs[i] * ym[j] for j in range(H))
                    for i in range(CH))
            init = tuple(
                tuple(y_ref[(nc + i) * H + j] for j in range(H))
                for i in range(CH))
            acc = lax.fori_loop(0, N, inner, init)
            for i in range(CH):
                dn = dinv_ref[nc + i]
                for j in range(H):
                    v = dn * acc[i][j] + bias_ref[j]
                    if relu:
                        v = jnp.maximum(v, 0.0)
                    dst_ref[(nc + i) * H + j] = v

    conv(h_ref, b1_ref, True)

    for n in range(N):
        hk = [h_ref[n * H + k] for k in range(H)]
        dn = dinv_ref[n]
        for j in range(H):
            hp = hk[0] * W2_ref[0, j]
            for k in range(1, H):
                hp = hp + hk[k] * W2_ref[k, j]
            y_ref[n * H + j] = dn * hp

    conv(h_ref, b2_ref, False)

    for j in range(H):
        s = h_ref[j]
        for n in range(1, N):
            s = s + h_ref[n * H + j]
        z_ref[j] = s * (1.0 / N)

    ubases = ([UI0 + f * (N + 1) for f in range(NF)]
              + [UT0 + f * (N + 1) for f in range(NF)])
    for pc in range(0, 2 * NF, CH):
        bases = [ubases[pc + i] for i in range(CH)]
        def innerU(v, acc, bases=bases):
            hv = [h_ref[(v - 1) * H + j] for j in range(H)]
            us = [T_ref[bases[i] + v] for i in range(CH)]
            return tuple(
                tuple(acc[i][j] + us[i] * hv[j] for j in range(H))
                for i in range(CH))
        acc = lax.fori_loop(1, N + 1, innerU,
                            tuple(tuple(zero for _ in range(H))
                                  for _ in range(CH)))
        for i in range(CH):
            for j in range(H):
                z_ref[(1 + pc + i) * H + j] = acc[i][j]

    def mlp_body(r, accm):
        zr = [z_ref[r * H + k] for k in range(H)]
        t1 = []
        for j in range(H):
            s = zr[0] * Wn1_ref[0, j]
            for k in range(1, H):
                s = s + zr[k] * Wn1_ref[k, j]
            t1.append(jnp.maximum(s + bn1_ref[j], 0.0))
        out = []
        for j in range(H):
            s = t1[0] * Wn2_ref[0, j]
            for k in range(1, H):
                s = s + t1[k] * Wn2_ref[k, j]
            out.append(accm[j] + s + bn2_ref[j])
        return tuple(out)
    accm = lax.fori_loop(0, ZROWS, mlp_body, tuple(zero for _ in range(H)))
    zm = [a * (1.0 / ZROWS) for a in accm]

    a1 = []
    for j in range(H):
        s = zm[0] * Wa1_ref[0, j]
        for k in range(1, H):
            s = s + zm[k] * Wa1_ref[k, j]
        a1.append(jnp.maximum(s + ba1_ref[j], 0.0))
    for p in range(NF):
        s = a1[0] * Wa2_ref[0, p]
        for k in range(1, H):
            s = s + a1[k] * Wa2_ref[k, p]
        out_ref[p] = s + ba2_ref[p]
    s = zm[0] * Wc_ref[0, 0]
    for k in range(1, H):
        s = s + zm[k] * Wc_ref[k, 0]
    out_ref[NF] = s + bc_ref[0]


def _run_tc(T2, xt2, W1, b1, W2, b2, Wn1, bn1, Wn2, bn2,
            Wa1, ba1, Wa2, ba2, Wc, bc, interpret=False):
    Bg = T2.shape[1]
    nblk = Bg // 1024
    T3 = T2.reshape(TW, Bg // 128, 128)
    xt3 = xt2.reshape(N * FIN, Bg // 128, 128)

    def vspec(rows):
        return pl.BlockSpec((rows, 8, 128), lambda i: (0, i, 0))
    sspec = pl.BlockSpec(memory_space=pltpu.SMEM)

    out = pl.pallas_call(
        _tc_kernel,
        grid=(nblk,),
        in_specs=[vspec(TW), vspec(N * FIN)] + [sspec] * 14,
        out_specs=pl.BlockSpec((NF + 1, 8, 128), lambda i: (0, i, 0)),
        out_shape=jax.ShapeDtypeStruct((NF + 1, Bg // 128, 128), jnp.float32),
        scratch_shapes=[
            pltpu.VMEM((N, 8, 128), jnp.float32),
            pltpu.VMEM((N * H, 8, 128), jnp.float32),
            pltpu.VMEM((N * H, 8, 128), jnp.float32),
            pltpu.VMEM((ZROWS * H, 8, 128), jnp.float32),
        ],
        interpret=interpret,
    )(T3, xt3, W1, b1, W2, b2, Wn1, bn1, Wn2, bn2,
      Wa1, ba1, Wa2, ba2, Wc, bc)
    out2 = out.reshape(NF + 1, Bg)
    return out2[:NF].T, out2[NF:NF + 1].T


def kernel(node_features, edge_index, init_config, target_config,
           W1, b1, W2, b2, Wn1, bn1, Wn2, bn2, Wa1, ba1, Wa2, ba2, Wc, bc):
    Bg = node_features.shape[0]
    Tf = _sc_build_tables(edge_index, init_config, target_config)
    T2 = Tf.reshape(Bg, TW).T
    xt2 = node_features.reshape(Bg, N * FIN).T
    logits, value = _run_tc(T2, xt2, W1, b1, W2, b2,
                            Wn1, bn1, Wn2, bn2, Wa1, ba1, Wa2, ba2, Wc, bc)
    return (logits, value)


# speedup vs baseline: 184.9748x; 1.0729x over previous
import functools

import jax
import jax.numpy as jnp
from jax import lax
from jax.experimental import pallas as pl
from jax.experimental.pallas import tpu as pltpu
from jax.experimental.pallas import tpu_sc as plsc

N = 30
E = 120
NF = 9
L = 10
FIN = 3
H = 12
UROWS = NF * (N + 1)
ZROWS = 2 * NF + 1
CWP = N * N + 1
UI0 = CWP
UT0 = CWP + UROWS
TW = CWP + 2 * UROWS
EW = 2 * E
FW = NF * L


def _sc_build_tables(edge_index, init_config, target_config):
    Bg = edge_index.shape[0]
    info = plsc.get_sparse_core_info()
    NC, NS, LN = info.num_cores, info.num_subcores, info.num_lanes
    NW = NC * NS
    gper = Bg // NW
    SGRP = 4
    rounds = gper // (SGRP * LN)
    mesh = plsc.VectorSubcoreMesh(core_axis_name="c", subcore_axis_name="s")

    SLOT = LN * TW
    NSAVE = E + 2 * NF * L

    @functools.partial(
        pl.kernel, mesh=mesh,
        compiler_params=pltpu.CompilerParams(needs_layout_passes=False),
        out_type=jax.ShapeDtypeStruct((Bg * TW,), jnp.float32),
        scratch_types=[
            pltpu.VMEM((SGRP * LN * EW,), jnp.int32),
            pltpu.VMEM((SGRP * LN * FW,), jnp.int32),
            pltpu.VMEM((SGRP * LN * FW,), jnp.int32),
            pltpu.VMEM((2 * SLOT,), jnp.float32),
            pltpu.VMEM((2 * NSAVE * LN,), jnp.int32),
        ],
    )
    def build(edge_hbm, init_hbm, targ_hbm, T_hbm,
              ebuf, ibuf, tbuf, work, abuf):
        i32 = jnp.int32
        f32 = jnp.float32
        iota = lax.iota(i32, LN)
        ones = jnp.ones((LN,), f32)
        zf = jnp.zeros((LN,), f32)
        iota_t = iota * TW
        iota_e = iota * EW
        iota_f = iota * FW
        wid = lax.axis_index("s") * NC + lax.axis_index("c")
        w0 = wid * gper

        @plsc.parallel_loop(0, TW, unroll=8)
        def z0(r):
            plsc.store_scatter(work, [iota_t + r], zf)
            plsc.store_scatter(work, [iota_t + (SLOT + r)], zf)

        def round_body(rd, _):
            r0 = w0 + rd * (SGRP * LN)
            pltpu.sync_copy(edge_hbm.at[pl.ds(r0 * EW, SGRP * LN * EW)], ebuf)
            pltpu.sync_copy(init_hbm.at[pl.ds(r0 * FW, SGRP * LN * FW)], ibuf)
            pltpu.sync_copy(targ_hbm.at[pl.ds(r0 * FW, SGRP * LN * FW)], tbuf)

            for gg in range(SGRP):
                s = gg % 2
                sbase = s * SLOT
                abase = s * NSAVE * LN
                se = gg * LN * EW
                sf = gg * LN * FW

                @plsc.parallel_loop(0, E, unroll=8)
                def ed(e, sbase=sbase, abase=abase, se=se):
                    eg = iota + e
                    eg = jnp.where(eg >= E, eg - E, eg)
                    vsrc = plsc.load_gather(ebuf, [iota_e + (se + eg)])
                    vdst = plsc.load_gather(
                        ebuf, [iota_e + (se + E + eg)])
                    addr = iota_t + (sbase + (vdst * N + vsrc))
                    plsc.addupdate_scatter(work, [addr], ones)
                    plsc.store_scatter(
                        abuf, [iota + (abase + e * LN)], addr)

                for f in range(NF):
                    @plsc.parallel_loop(0, L, unroll=5)
                    def ll(l, f=f, sbase=sbase, abase=abase, sf=sf):
                        for ci, cfg in ((0, ibuf), (1, tbuf)):
                            ub = sbase + UI0 + ci * UROWS + f * (N + 1)
                            asl = abase + (E + ci * FW + f * L) * LN
                            v = plsc.load_gather(
                                cfg, [iota_f + (sf + f * L + l)])
                            addr = iota_t + (ub + v)
                            plsc.addupdate_scatter(work, [addr], ones)
                            plsc.store_scatter(
                                abuf, [iota + (asl + l * LN)], addr)

                if gg % 2 == 1:
                    b0 = r0 + (gg - 1) * LN
                    pltpu.sync_copy(
                        work, T_hbm.at[pl.ds(b0 * TW, 2 * SLOT)])
                    @plsc.parallel_loop(0, 2 * NSAVE, unroll=8)
                    def rz(t):
                        av = plsc.load_gather(abuf, [iota + t * LN])
                        plsc.store_scatter(work, [av], zf)
            return ()

        lax.fori_loop(0, rounds, round_body, ())

    return build(edge_index.reshape(Bg * EW),
                 init_config.reshape(Bg * FW),
                 target_config.reshape(Bg * FW))


def _tc_kernel(T_ref, xt_ref,
               W1_ref, b1_ref, W2_ref, b2_ref,
               Wn1_ref, bn1_ref, Wn2_ref, bn2_ref,
               Wa1_ref, ba1_ref, Wa2_ref, ba2_ref, Wc_ref, bc_ref,
               out_ref, dinv_ref, y_ref, h_ref, z_ref):
    f32 = jnp.float32
    zero = jnp.zeros((8, 128), f32)
    CH = 3

    for n in range(N):
        s = T_ref[n * N]
        for m in range(1, N):
            s = s + T_ref[n * N + m]
        dinv_ref[n] = 1.0 / jnp.sqrt(s + 1.0)

    for n in range(N):
        xk = [xt_ref[n * FIN + k] for k in range(FIN)]
        dn = dinv_ref[n]
        for j in range(H):
            hp = xk[0] * W1_ref[0, j]
            for k in range(1, FIN):
                hp = hp + xk[k] * W1_ref[k, j]
            y_ref[n * H + j] = dn * hp

    def conv(dst_ref, bias_ref, relu):
        for nc in range(0, N, CH):
            def inner(m, acc, nc=nc):
                ym = [y_ref[m * H + j] for j in range(H)]
                cs = [T_ref[(nc + i) * N + m] for i in range(CH)]
                return tuple(
                    tuple(acc[i][j] + cs[i] * ym[j] for j in range(H))
                    for i in range(CH))
            init = tuple(
                tuple(y_ref[(nc + i) * H + j] for j in range(H))
                for i in range(CH))
            acc = lax.fori_loop(0, N, inner, init)
            for i in range(CH):
                dn = dinv_ref[nc + i]
                for j in range(H):
                    v = dn * acc[i][j] + bias_ref[j]
                    if relu:
                        v = jnp.maximum(v, 0.0)
                    dst_ref[(nc + i) * H + j] = v

    conv(h_ref, b1_ref, True)

    for n in range(N):
        hk = [h_ref[n * H + k] for k in range(H)]
        dn = dinv_ref[n]
        for j in range(H):
            hp = hk[0] * W2_ref[0, j]
            for k in range(1, H):
                hp = hp + hk[k] * W2_ref[k, j]
            y_ref[n * H + j] = dn * hp

    conv(h_ref, b2_ref, False)

    for j in range(H):
        s = h_ref[j]
        for n in range(1, N):
            s = s + h_ref[n * H + j]
        z_ref[j] = s * (1.0 / N)

    ubases = ([UI0 + f * (N + 1) for f in range(NF)]
              + [UT0 + f * (N + 1) for f in range(NF)])
    for pc in range(0, 2 * NF, CH):
        bases = [ubases[pc + i] for i in range(CH)]
        def innerU(v, acc, bases=bases):
            hv = [h_ref[(v - 1) * H + j] for j in range(H)]
            us = [T_ref[bases[i] + v] for i in range(CH)]
            return tuple(
                tuple(acc[i][j] + us[i] * hv[j] for j in range(H))
                for i in range(CH))
        acc = lax.fori_loop(1, N + 1, innerU,
                            tuple(tuple(zero for _ in range(H))
                                  for _ in range(CH)))
        for i in range(CH):
            for j in range(H):
                z_ref[(1 + pc + i) * H + j] = acc[i][j]

    def mlp_body(r, accm):
        zr = [z_ref[r * H + k] for k in range(H)]
        t1 = []
        for j in range(H):
            s = zr[0] * Wn1_ref[0, j]
            for k in range(1, H):
                s = s + zr[k] * Wn1_ref[k, j]
            t1.append(jnp.maximum(s + bn1_ref[j], 0.0))
        out = []
        for j in range(H):
            s = t1[0] * Wn2_ref[0, j]
            for k in range(1, H):
                s = s + t1[k] * Wn2_ref[k, j]
            out.append(accm[j] + s + bn2_ref[j])
        return tuple(out)
    accm = lax.fori_loop(0, ZROWS, mlp_body, tuple(zero for _ in range(H)))
    zm = [a * (1.0 / ZROWS) for a in accm]

    a1 = []
    for j in range(H):
        s = zm[0] * Wa1_ref[0, j]
        for k in range(1, H):
            s = s + zm[k] * Wa1_ref[k, j]
        a1.append(jnp.maximum(s + ba1_ref[j], 0.0))
    for p in range(NF):
        s = a1[0] * Wa2_ref[0, p]
        for k in range(1, H):
            s = s + a1[k] * Wa2_ref[k, p]
        out_ref[p] = s + ba2_ref[p]
    s = zm[0] * Wc_ref[0, 0]
    for k in range(1, H):
        s = s + zm[k] * Wc_ref[k, 0]
    out_ref[NF] = s + bc_ref[0]


def _run_tc(T2, xt2, W1, b1, W2, b2, Wn1, bn1, Wn2, bn2,
            Wa1, ba1, Wa2, ba2, Wc, bc, interpret=False):
    Bg = T2.shape[1]
    nblk = Bg // 1024
    T3 = T2.reshape(TW, Bg // 128, 128)
    xt3 = xt2.reshape(N * FIN, Bg // 128, 128)

    def vspec(rows):
        return pl.BlockSpec((rows, 8, 128), lambda i: (0, i, 0))
    sspec = pl.BlockSpec(memory_space=pltpu.SMEM)

    out = pl.pallas_call(
        _tc_kernel,
        grid=(nblk,),
        in_specs=[vspec(TW), vspec(N * FIN)] + [sspec] * 14,
        out_specs=pl.BlockSpec((NF + 1, 8, 128), lambda i: (0, i, 0)),
        out_shape=jax.ShapeDtypeStruct((NF + 1, Bg // 128, 128), jnp.float32),
        scratch_shapes=[
            pltpu.VMEM((N, 8, 128), jnp.float32),
            pltpu.VMEM((N * H, 8, 128), jnp.float32),
            pltpu.VMEM((N * H, 8, 128), jnp.float32),
            pltpu.VMEM((ZROWS * H, 8, 128), jnp.float32),
        ],
        interpret=interpret,
    )(T3, xt3, W1, b1, W2, b2, Wn1, bn1, Wn2, bn2,
      Wa1, ba1, Wa2, ba2, Wc, bc)
    out2 = out.reshape(NF + 1, Bg)
    return out2[:NF].T, out2[NF:NF + 1].T


def kernel(node_features, edge_index, init_config, target_config,
           W1, b1, W2, b2, Wn1, bn1, Wn2, bn2, Wa1, ba1, Wa2, ba2, Wc, bc):
    Bg = node_features.shape[0]
    Tf = _sc_build_tables(edge_index, init_config, target_config)
    T2 = Tf.reshape(Bg, TW).T
    xt2 = node_features.reshape(Bg, N * FIN).T
    logits, value = _run_tc(T2, xt2, W1, b1, W2, b2,
                            Wn1, bn1, Wn2, bn2, Wa1, ba1, Wa2, ba2, Wc, bc)
    return (logits, value)


# speedup vs baseline: 189.1391x; 1.0225x over previous
import functools

import jax
import jax.numpy as jnp
from jax import lax
from jax.experimental import pallas as pl
from jax.experimental.pallas import tpu as pltpu
from jax.experimental.pallas import tpu_sc as plsc

N = 30
E = 120
NF = 9
L = 10
FIN = 3
H = 12
UROWS = NF * (N + 1)
ZROWS = 2 * NF + 1
CWP = N * N + 1
UI0 = CWP
UT0 = CWP + UROWS
TW = CWP + 2 * UROWS
EW = 2 * E
FW = NF * L


def _sc_build_tables(edge_index, init_config, target_config):
    Bg = edge_index.shape[0]
    info = plsc.get_sparse_core_info()
    NC, NS, LN = info.num_cores, info.num_subcores, info.num_lanes
    NW = NC * NS
    gper = Bg // NW
    SGRP = 4
    rounds = gper // (SGRP * LN)
    mesh = plsc.VectorSubcoreMesh(core_axis_name="c", subcore_axis_name="s")

    SLOT = LN * TW
    NSAVE = E + 2 * NF * L

    @functools.partial(
        pl.kernel, mesh=mesh,
        compiler_params=pltpu.CompilerParams(needs_layout_passes=False),
        out_type=jax.ShapeDtypeStruct((Bg * TW,), jnp.float32),
        scratch_types=[
            pltpu.VMEM((SGRP * LN * EW,), jnp.int32),
            pltpu.VMEM((SGRP * LN * FW,), jnp.int32),
            pltpu.VMEM((SGRP * LN * FW,), jnp.int32),
            pltpu.VMEM((2 * SLOT,), jnp.float32),
            pltpu.VMEM((2 * NSAVE * LN,), jnp.int32),
        ],
    )
    def build(edge_hbm, init_hbm, targ_hbm, T_hbm,
              ebuf, ibuf, tbuf, work, abuf):
        i32 = jnp.int32
        f32 = jnp.float32
        iota = lax.iota(i32, LN)
        ones = jnp.ones((LN,), f32)
        zf = jnp.zeros((LN,), f32)
        iota_t = iota * TW
        iota_e = iota * EW
        iota_f = iota * FW
        wid = lax.axis_index("s") * NC + lax.axis_index("c")
        w0 = wid * gper

        @plsc.parallel_loop(0, TW, unroll=8)
        def z0(r):
            plsc.store_scatter(work, [iota_t + r], zf)
            plsc.store_scatter(work, [iota_t + (SLOT + r)], zf)

        def round_body(rd, _):
            r0 = w0 + rd * (SGRP * LN)
            pltpu.sync_copy(edge_hbm.at[pl.ds(r0 * EW, SGRP * LN * EW)], ebuf)
            pltpu.sync_copy(init_hbm.at[pl.ds(r0 * FW, SGRP * LN * FW)], ibuf)
            pltpu.sync_copy(targ_hbm.at[pl.ds(r0 * FW, SGRP * LN * FW)], tbuf)

            for gg in range(SGRP):
                s = gg % 2
                sbase = s * SLOT
                abase = s * NSAVE * LN
                se = gg * LN * EW
                sf = gg * LN * FW

                @plsc.parallel_loop(0, E, unroll=8)
                def ed(e, sbase=sbase, abase=abase, se=se):
                    eg = iota + e
                    eg = jnp.where(eg >= E, eg - E, eg)
                    vsrc = plsc.load_gather(ebuf, [iota_e + (se + eg)])
                    vdst = plsc.load_gather(
                        ebuf, [iota_e + (se + E + eg)])
                    addr = iota_t + (sbase + (vdst * N + vsrc))
                    plsc.addupdate_scatter(work, [addr], ones)
                    plsc.store_scatter(
                        abuf, [iota + (abase + e * LN)], addr)

                for f in range(NF):
                    @plsc.parallel_loop(0, L, unroll=5)
                    def ll(l, f=f, sbase=sbase, abase=abase, sf=sf):
                        for ci, cfg in ((0, ibuf), (1, tbuf)):
                            ub = sbase + UI0 + ci * UROWS + f * (N + 1)
                            asl = abase + (E + ci * FW + f * L) * LN
                            v = plsc.load_gather(
                                cfg, [iota_f + (sf + f * L + l)])
                            addr = iota_t + (ub + v)
                            plsc.addupdate_scatter(work, [addr], ones)
                            plsc.store_scatter(
                                abuf, [iota + (asl + l * LN)], addr)

                if gg % 2 == 1:
                    b0 = r0 + (gg - 1) * LN
                    pltpu.sync_copy(
                        work, T_hbm.at[pl.ds(b0 * TW, 2 * SLOT)])
                    @plsc.parallel_loop(0, 2 * NSAVE, unroll=8)
                    def rz(t):
                        av = plsc.load_gather(abuf, [iota + t * LN])
                        plsc.store_scatter(work, [av], zf)
            return ()

        lax.fori_loop(0, rounds, round_body, ())

    return build(edge_index.reshape(Bg * EW),
                 init_config.reshape(Bg * FW),
                 target_config.reshape(Bg * FW))


def _tc_kernel(T_ref, xt_ref,
               W1_ref, b1_ref, W2_ref, b2_ref,
               Wn1_ref, bn1_ref, Wn2_ref, bn2_ref,
               Wa1_ref, ba1_ref, Wa2_ref, ba2_ref, Wc_ref, bc_ref,
               out_ref, dinv_ref, y_ref, h_ref, z_ref):
    f32 = jnp.float32
    zero = jnp.zeros((8, 128), f32)
    CH = 3

    for n in range(N):
        s = T_ref[n * N]
        for m in range(1, N):
            s = s + T_ref[n * N + m]
        dinv_ref[n] = 1.0 / jnp.sqrt(s + 1.0)

    for n in range(N):
        xk = [xt_ref[n * FIN + k] for k in range(FIN)]
        dn = dinv_ref[n]
        for j in range(H):
            hp = xk[0] * W1_ref[0, j]
            for k in range(1, FIN):
                hp = hp + xk[k] * W1_ref[k, j]
            y_ref[n * H + j] = dn * hp

    def conv(dst_ref, bias_ref, relu):
        for nc in range(0, N, CH):
            def inner(m, acc, nc=nc):
                ym = [y_ref[m * H + j] for j in range(H)]
                cs = [T_ref[(nc + i) * N + m] for i in range(CH)]
                return tuple(
                    tuple(acc[i][j] + cs[i] * ym[j] for j in range(H))
                    for i in range(CH))
            init = tuple(
                tuple(y_ref[(nc + i) * H + j] for j in range(H))
                for i in range(CH))
            acc = lax.fori_loop(0, N, inner, init)
            for i in range(CH):
                dn = dinv_ref[nc + i]
                for j in range(H):
                    v = dn * acc[i][j] + bias_ref[j]
                    if relu:
                        v = jnp.maximum(v, 0.0)
                    dst_ref[(nc + i) * H + j] = v

    conv(h_ref, b1_ref, True)

    for n in range(N):
        hk = [h_ref[n * H + k] for k in range(H)]
        dn = dinv_ref[n]
        for j in range(H):
            hp = hk[0] * W2_ref[0, j]
            for k in range(1, H):
                hp = hp + hk[k] * W2_ref[k, j]
            y_ref[n * H + j] = dn * hp

    conv(h_ref, b2_ref, False)

    for j in range(H):
        s = h_ref[j]
        for n in range(1, N):
            s = s + h_ref[n * H + j]
        z_ref[j] = s * (1.0 / N)

    ubases = ([UI0 + f * (N + 1) for f in range(NF)]
              + [UT0 + f * (N + 1) for f in range(NF)])
    for pc in range(0, 2 * NF, CH):
        bases = [ubases[pc + i] for i in range(CH)]
        def innerU(v, acc, bases=bases):
            hv = [h_ref[(v - 1) * H + j] for j in range(H)]
            us = [T_ref[bases[i] + v] for i in range(CH)]
            return tuple(
                tuple(acc[i][j] + us[i] * hv[j] for j in range(H))
                for i in range(CH))
        acc = lax.fori_loop(1, N + 1, innerU,
                            tuple(tuple(zero for _ in range(H))
                                  for _ in range(CH)))
        for i in range(CH):
            for j in range(H):
                z_ref[(1 + pc + i) * H + j] = acc[i][j]

    def mlp_body(r, accm):
        zr = [z_ref[r * H + k] for k in range(H)]
        t1 = []
        for j in range(H):
            s = zr[0] * Wn1_ref[0, j]
            for k in range(1, H):
                s = s + zr[k] * Wn1_ref[k, j]
            t1.append(jnp.maximum(s + bn1_ref[j], 0.0))
        out = []
        for j in range(H):
            s = t1[0] * Wn2_ref[0, j]
            for k in range(1, H):
                s = s + t1[k] * Wn2_ref[k, j]
            out.append(accm[j] + s + bn2_ref[j])
        return tuple(out)
    accm = lax.fori_loop(0, ZROWS, mlp_body, tuple(zero for _ in range(H)))
    zm = [a * (1.0 / ZROWS) for a in accm]

    a1 = []
    for j in range(H):
        s = zm[0] * Wa1_ref[0, j]
        for k in range(1, H):
            s = s + zm[k] * Wa1_ref[k, j]
        a1.append(jnp.maximum(s + ba1_ref[j], 0.0))
    for p in range(NF):
        s = a1[0] * Wa2_ref[0, p]
        for k in range(1, H):
            s = s + a1[k] * Wa2_ref[k, p]
        out_ref[p] = s + ba2_ref[p]
    s = zm[0] * Wc_ref[0, 0]
    for k in range(1, H):
        s = s + zm[k] * Wc_ref[k, 0]
    out_ref[NF] = s + bc_ref[0]


def _run_tc(T2, xt2, W1, b1, W2, b2, Wn1, bn1, Wn2, bn2,
            Wa1, ba1, Wa2, ba2, Wc, bc, interpret=False):
    Bg = T2.shape[1]
    nblk = Bg // 1024
    T3 = T2.reshape(TW, Bg // 128, 128)
    xt3 = xt2.reshape(N * FIN, Bg // 128, 128)

    def vspec(rows):
        return pl.BlockSpec((rows, 8, 128), lambda i: (0, i, 0))
    sspec = pl.BlockSpec(memory_space=pltpu.SMEM)

    out = pl.pallas_call(
        _tc_kernel,
        grid=(nblk,),
        in_specs=[vspec(TW), vspec(N * FIN)] + [sspec] * 14,
        out_specs=pl.BlockSpec((NF + 1, 8, 128), lambda i: (0, i, 0)),
        out_shape=jax.ShapeDtypeStruct((NF + 1, Bg // 128, 128), jnp.float32),
        scratch_shapes=[
            pltpu.VMEM((N, 8, 128), jnp.float32),
            pltpu.VMEM((N * H, 8, 128), jnp.float32),
            pltpu.VMEM((N * H, 8, 128), jnp.float32),
            pltpu.VMEM((ZROWS * H, 8, 128), jnp.float32),
        ],
        interpret=interpret,
    )(T3, xt3, W1, b1, W2, b2, Wn1, bn1, Wn2, bn2,
      Wa1, ba1, Wa2, ba2, Wc, bc)
    out2 = out.reshape(NF + 1, Bg)
    return out2[:NF].T, out2[NF:NF + 1].T


def kernel(node_features, edge_index, init_config, target_config,
           W1, b1, W2, b2, Wn1, bn1, Wn2, bn2, Wa1, ba1, Wa2, ba2, Wc, bc):
    Bg = node_features.shape[0]
    Hh = Bg // 2
    outs = []
    for h0 in (0, Hh):
        Tf = _sc_build_tables(edge_index[h0:h0 + Hh],
                              init_config[h0:h0 + Hh],
                              target_config[h0:h0 + Hh])
        T2 = Tf.reshape(Hh, TW).T
        xt2 = node_features[h0:h0 + Hh].reshape(Hh, N * FIN).T
        outs.append(_run_tc(T2, xt2, W1, b1, W2, b2, Wn1, bn1, Wn2, bn2,
                            Wa1, ba1, Wa2, ba2, Wc, bc))
    logits = jnp.concatenate([outs[0][0], outs[1][0]], axis=0)
    value = jnp.concatenate([outs[0][1], outs[1][1]], axis=0)
    return (logits, value)
